# Initial kernel scaffold; baseline (speedup 1.0000x reference)
import jax
import jax.numpy as jnp
from jax import lax
from jax.experimental import pallas as pl
from jax.experimental.pallas import tpu as pltpu
from jax.experimental.pallas import tpu_sc as plsc

_N = 10000
_E = 320000
_D = 128
_G = 64
_NC = 2
_NS = 16
_NW = _NC * _NS
_CH = 128
_NCHUNK = 80
_EPW = _CH * _NCHUNK
_EPAD = _EPW * _NW
_NP = 10240
_NBUF = 4
_RB = 1024
_GRID = _NP // _RB

_mesh = plsc.VectorSubcoreMesh(
    core_axis_name="c", subcore_axis_name="s", num_cores=_NC, num_subcores=_NS
)


def _deg_body(dst_hbm, out_hbm, idx_v, acc):
    c = lax.axis_index("c")
    s = lax.axis_index("s")
    wid = c * _NS + s
    pltpu.sync_copy(dst_hbm.at[wid], idx_v)

    @pl.loop(0, _NP // 16)
    def _zero(i):
        acc[pl.ds(i * 16, 16)] = jnp.zeros((16,), jnp.float32)

    ones = jnp.ones((16,), jnp.float32)

    @pl.loop(0, _EPW // 16)
    def _count(i):
        idx = idx_v[pl.ds(i * 16, 16)]
        plsc.addupdate_scatter(acc, [idx], ones)

    pltpu.sync_copy(acc, out_hbm.at[wid])


_deg_kernel = pl.kernel(
    _deg_body,
    out_type=jax.ShapeDtypeStruct((_NW, _NP), jnp.float32),
    mesh=_mesh,
    scratch_types=[
        pltpu.VMEM((_EPW,), jnp.int32),
        pltpu.VMEM((_NP,), jnp.float32),
    ],
)


def _agg_body(z_hbm, src_hbm, dst_hbm, out_hbm, sidx, didx, rows, zb, acc, sems):
    c = lax.axis_index("c")
    s = lax.axis_index("s")
    wid = c * _NS + s
    pltpu.sync_copy(src_hbm.at[wid], sidx)
    pltpu.sync_copy(dst_hbm.at[wid], didx)

    for b in range(_NBUF):
        pltpu.async_copy(z_hbm.at[sidx.at[b]], rows.at[b], sems.at[b])

    @pl.loop(0, 64)
    def _zero(j):
        for k in range(8):
            zb[j, pl.ds(k * 16, 16)] = jnp.zeros((16,), jnp.float32)

    rpt = _NP // _NS
    for t in range(rpt // 64):
        pltpu.sync_copy(zb, acc.at[pl.ds(s * rpt + t * 64, 64)])
    plsc.subcore_barrier()

    @pl.loop(0, _NCHUNK - _NBUF, step=_NBUF)
    def _run(i):
        for b in range(_NBUF):
            j = i + b
            pltpu.make_async_copy(z_hbm.at[sidx.at[j]], rows.at[b], sems.at[b]).wait()
            pltpu.sync_copy(rows.at[b], acc.at[didx.at[j]], add=True)
            pltpu.async_copy(z_hbm.at[sidx.at[j + _NBUF]], rows.at[b], sems.at[b])

    for b in range(_NBUF):
        j = _NCHUNK - _NBUF + b
        pltpu.make_async_copy(z_hbm.at[sidx.at[j]], rows.at[b], sems.at[b]).wait()
        pltpu.sync_copy(rows.at[b], acc.at[didx.at[j]], add=True)
    plsc.subcore_barrier()

    base = s * rpt
    for t in range(rpt // _CH):
        pltpu.sync_copy(acc.at[pl.ds(base + t * _CH, _CH)], rows.at[0])
        pltpu.sync_copy(rows.at[0], out_hbm.at[c, pl.ds(base + t * _CH, _CH)])


_agg_kernel = pl.kernel(
    _agg_body,
    out_type=jax.ShapeDtypeStruct((_NC, _NP, _D), jnp.float32),
    mesh=_mesh,
    scratch_types=[
        pltpu.VMEM((_NCHUNK, _CH), jnp.int32),
        pltpu.VMEM((_NCHUNK, _CH), jnp.int32),
        pltpu.VMEM((_NBUF, _CH, _D), jnp.float32),
        pltpu.VMEM((64, _D), jnp.float32),
        pltpu.VMEM_SHARED((_NP, _D), jnp.float32),
        pltpu.SemaphoreType.DMA((_NBUF,)),
    ],
)


def _dis_of(dp_ref):
    deg = jnp.sum(dp_ref[...], axis=0) + 1.0
    return lax.rsqrt(deg)


def _mm1_body(dp_ref, x_ref, w_ref, o_ref):
    dis = _dis_of(dp_ref)
    z = jnp.dot(x_ref[...], w_ref[...], preferred_element_type=jnp.float32)
    o_ref[...] = z * dis[:, None]


def _mid_body(dp_ref, p_ref, z_ref, b_ref, w_ref, o_ref):
    dis = _dis_of(dp_ref)
    agg = p_ref[0] + p_ref[1] + z_ref[...]
    h = jnp.maximum(agg * dis[:, None] + b_ref[...], 0.0)
    o_ref[...] = jnp.dot(h, w_ref[...], preferred_element_type=jnp.float32) * dis[:, None]


def _pool_body(dp_ref, p_ref, z_ref, b_ref, bat_ref, o_ref, sums, cnts):
    i = pl.program_id(0)
    dis = _dis_of(dp_ref)
    h = (p_ref[0] + p_ref[1] + z_ref[...]) * dis[:, None] + b_ref[...]
    ids = lax.broadcasted_iota(jnp.float32, (_RB, _G), 1)
    oh = (bat_ref[...] == ids).astype(jnp.float32)
    bsum = lax.dot_general(oh, h, (((0,), (0,)), ((), ())),
                           preferred_element_type=jnp.float32)
    bcnt = jnp.sum(oh, axis=0)[:, None] * jnp.ones((1, _D), jnp.float32)

    @pl.when(i == 0)
    def _init():
        sums[...] = jnp.zeros_like(sums)
        cnts[...] = jnp.zeros_like(cnts)

    sums[...] += bsum
    cnts[...] += bcnt

    @pl.when(i == _GRID - 1)
    def _fin():
        o_ref[...] = sums[...] / jnp.maximum(cnts[...], 1.0)


_spec_dp = pl.BlockSpec((_NW, _RB), lambda i: (0, i))
_spec_row = pl.BlockSpec((_RB, _D), lambda i: (i, 0))
_spec_p = pl.BlockSpec((_NC, _RB, _D), lambda i: (0, i, 0))
_spec_w = pl.BlockSpec((_D, _D), lambda i: (0, 0))
_spec_b = pl.BlockSpec((1, _D), lambda i: (0, 0))
_spec_bat = pl.BlockSpec((_RB, 1), lambda i: (i, 0))

_mm1 = pl.pallas_call(
    _mm1_body,
    grid=(_GRID,),
    in_specs=[_spec_dp, _spec_row, _spec_w],
    out_specs=_spec_row,
    out_shape=jax.ShapeDtypeStruct((_NP, _D), jnp.float32),
)

_mid = pl.pallas_call(
    _mid_body,
    grid=(_GRID,),
    in_specs=[_spec_dp, _spec_p, _spec_row, _spec_b, _spec_w],
    out_specs=_spec_row,
    out_shape=jax.ShapeDtypeStruct((_NP, _D), jnp.float32),
)

_pool = pl.pallas_call(
    _pool_body,
    grid=(_GRID,),
    in_specs=[_spec_dp, _spec_p, _spec_row, _spec_b, _spec_bat],
    out_specs=pl.BlockSpec((_G, _D), lambda i: (0, 0)),
    out_shape=jax.ShapeDtypeStruct((_G, _D), jnp.float32),
    scratch_shapes=[
        pltpu.VMEM((_G, _D), jnp.float32),
        pltpu.VMEM((_G, _D), jnp.float32),
    ],
)


@jax.jit
def kernel(x, edge_index, batch, W1, b1, W2, b2, W3, b3):
    src = edge_index[0].astype(jnp.int32)
    dst = edge_index[1].astype(jnp.int32)
    pad = _EPAD - _E
    src_p = jnp.concatenate([src, jnp.zeros((pad,), jnp.int32)])
    dst_p = jnp.concatenate([dst, jnp.full((pad,), _N, jnp.int32)])
    src3 = src_p.reshape(_NW, _NCHUNK, _CH)
    dst3 = dst_p.reshape(_NW, _NCHUNK, _CH)
    x_p = jnp.pad(x, ((0, _NP - _N), (0, 0)))
    bat = jnp.pad(batch.astype(jnp.int32), (0, _NP - _N), constant_values=_G)
    bat = bat.astype(jnp.float32).reshape(_NP, 1)
    b1r = b1.reshape(1, _D)
    b2r = b2.reshape(1, _D)
    b3r = b3.reshape(1, _D)

    deg_parts = _deg_kernel(dst_p.reshape(_NW, _EPW))
    z1 = _mm1(deg_parts, x_p, W1)
    p1 = _agg_kernel(z1, src3, dst3)
    z2 = _mid(deg_parts, p1, z1, b1r, W2)
    p2 = _agg_kernel(z2, src3, dst3)
    z3 = _mid(deg_parts, p2, z2, b2r, W3)
    p3 = _agg_kernel(z3, src3, dst3)
    return _pool(deg_parts, p3, z3, b3r, bat)


# speedup vs baseline: 12.1418x; 12.1418x over previous
import jax
import jax.numpy as jnp
from jax import lax
from jax.experimental import pallas as pl
from jax.experimental.pallas import tpu as pltpu
from jax.experimental.pallas import tpu_sc as plsc

_N = 10000
_E = 320000
_D = 128
_H = 64
_G = 64
_NC = 2
_NS = 16
_NW = _NC * _NS
_CH = 128
_NCH = 160
_EPT = _CH * _NCH
_EPAD = _EPT * _NS
_NP = 10240
_NBUF = 4
_RB = 1024
_GRID = _NP // _RB

_mesh = plsc.VectorSubcoreMesh(
    core_axis_name="c", subcore_axis_name="s", num_cores=_NC, num_subcores=_NS
)
_sc_params = pltpu.CompilerParams(
    needs_layout_passes=False, use_tc_tiling_on_sc=False
)


def _deg_body(dst_hbm, out_hbm, idx_v, acc):
    c = lax.axis_index("c")
    s = lax.axis_index("s")
    wid = c * _NS + s
    pltpu.sync_copy(dst_hbm.at[wid], idx_v)

    @pl.loop(0, _NP // 16)
    def _zero(i):
        acc[pl.ds(i * 16, 16)] = jnp.zeros((16,), jnp.float32)

    ones = jnp.ones((16,), jnp.float32)

    @pl.loop(0, (_EPAD // _NW) // 16)
    def _count(i):
        idx = idx_v[pl.ds(i * 16, 16)]
        plsc.addupdate_scatter(acc, [idx], ones)

    pltpu.sync_copy(acc, out_hbm.at[wid])


_deg_kernel = pl.kernel(
    _deg_body,
    out_type=jax.ShapeDtypeStruct((_NW, _NP), jnp.float32),
    mesh=_mesh,
    compiler_params=_sc_params,
    scratch_types=[
        pltpu.VMEM((_EPAD // _NW,), jnp.int32),
        pltpu.VMEM((_NP,), jnp.float32),
    ],
)


def _agg_body(z_hbm, src_hbm, dst_hbm, out_hbm, sidx, didx, rows, zb, acc, sems):
    c = lax.axis_index("c")
    s = lax.axis_index("s")
    zc = z_hbm.at[c]
    pltpu.sync_copy(src_hbm.at[s], sidx)
    pltpu.sync_copy(dst_hbm.at[s], didx)

    for b in range(_NBUF):
        pltpu.async_copy(zc.at[sidx.at[b]], rows.at[b], sems.at[b])

    @pl.loop(0, 64)
    def _zero(j):
        for k in range(_H // 16):
            zb[j, pl.ds(k * 16, 16)] = jnp.zeros((16,), jnp.float32)

    rpt = _NP // _NS
    for t in range(rpt // 64):
        pltpu.sync_copy(zb, acc.at[pl.ds(s * rpt + t * 64, 64)])
    plsc.subcore_barrier()

    @pl.loop(0, _NCH - _NBUF, step=_NBUF)
    def _run(i):
        for b in range(_NBUF):
            j = i + b
            pltpu.make_async_copy(zc.at[sidx.at[j]], rows.at[b], sems.at[b]).wait()
            pltpu.sync_copy(rows.at[b], acc.at[didx.at[j]], add=True)
            pltpu.async_copy(zc.at[sidx.at[j + _NBUF]], rows.at[b], sems.at[b])

    for b in range(_NBUF):
        j = _NCH - _NBUF + b
        pltpu.make_async_copy(zc.at[sidx.at[j]], rows.at[b], sems.at[b]).wait()
        pltpu.sync_copy(rows.at[b], acc.at[didx.at[j]], add=True)
    plsc.subcore_barrier()

    base = s * rpt
    for t in range(rpt // _CH):
        pltpu.sync_copy(acc.at[pl.ds(base + t * _CH, _CH)], rows.at[0])
        pltpu.sync_copy(rows.at[0], out_hbm.at[c, pl.ds(base + t * _CH, _CH)])


_agg_kernel = pl.kernel(
    _agg_body,
    out_type=jax.ShapeDtypeStruct((_NC, _NP, _H), jnp.float32),
    mesh=_mesh,
    compiler_params=_sc_params,
    scratch_types=[
        pltpu.VMEM((_NCH, _CH), jnp.int32),
        pltpu.VMEM((_NCH, _CH), jnp.int32),
        pltpu.VMEM((_NBUF, _CH, _H), jnp.float32),
        pltpu.VMEM((64, _H), jnp.float32),
        pltpu.VMEM_SHARED((_NP, _H), jnp.float32),
        pltpu.SemaphoreType.DMA((_NBUF,)),
    ],
)


def _dis_of(dp_ref):
    deg = jnp.sum(dp_ref[...], axis=0) + 1.0
    return lax.rsqrt(deg)


def _mm1_body(dp_ref, x_ref, wl_ref, wr_ref, o_ref):
    dis = _dis_of(dp_ref)[:, None]
    x = x_ref[...]
    o_ref[0] = jnp.dot(x, wl_ref[...], preferred_element_type=jnp.float32) * dis
    o_ref[1] = jnp.dot(x, wr_ref[...], preferred_element_type=jnp.float32) * dis


def _mid_body(dp_ref, pl_ref, pr_ref, z_ref, bl_ref, br_ref,
              q00_ref, q01_ref, q10_ref, q11_ref, o_ref):
    dis = _dis_of(dp_ref)[:, None]
    hl = jnp.maximum((pl_ref[0] + z_ref[0]) * dis + bl_ref[...], 0.0)
    hr = jnp.maximum((pr_ref[0] + z_ref[1]) * dis + br_ref[...], 0.0)
    dot = lambda a, b: jnp.dot(a, b, preferred_element_type=jnp.float32)
    o_ref[0] = (dot(hl, q00_ref[...]) + dot(hr, q10_ref[...])) * dis
    o_ref[1] = (dot(hl, q01_ref[...]) + dot(hr, q11_ref[...])) * dis


def _pool_body(dp_ref, pl_ref, pr_ref, z_ref, bl_ref, br_ref, bat_ref,
               o_ref, sums_l, sums_r, cnts):
    i = pl.program_id(0)
    dis = _dis_of(dp_ref)[:, None]
    hl = (pl_ref[0] + z_ref[0]) * dis + bl_ref[...]
    hr = (pr_ref[0] + z_ref[1]) * dis + br_ref[...]
    ids = lax.broadcasted_iota(jnp.int32, (_RB, _G), 1)
    oh = (bat_ref[...] == ids).astype(jnp.float32)
    dn = (((0,), (0,)), ((), ()))
    bsum_l = lax.dot_general(oh, hl, dn, preferred_element_type=jnp.float32)
    bsum_r = lax.dot_general(oh, hr, dn, preferred_element_type=jnp.float32)
    bcnt = jnp.sum(oh, axis=0)[:, None] * jnp.ones((1, _H), jnp.float32)

    @pl.when(i == 0)
    def _init():
        sums_l[...] = jnp.zeros_like(sums_l)
        sums_r[...] = jnp.zeros_like(sums_r)
        cnts[...] = jnp.zeros_like(cnts)

    sums_l[...] += bsum_l
    sums_r[...] += bsum_r
    cnts[...] += bcnt

    @pl.when(i == _GRID - 1)
    def _fin():
        c = jnp.maximum(cnts[...], 1.0)
        o_ref[0] = sums_l[...] / c
        o_ref[1] = sums_r[...] / c


_spec_dp = pl.BlockSpec((_NW, _RB), lambda i: (0, i))
_spec_x = pl.BlockSpec((_RB, _D), lambda i: (i, 0))
_spec_z = pl.BlockSpec((_NC, _RB, _H), lambda i: (0, i, 0))
_spec_pl = pl.BlockSpec((1, _RB, _H), lambda i: (0, i, 0))
_spec_pr = pl.BlockSpec((1, _RB, _H), lambda i: (1, i, 0))
_spec_wh = pl.BlockSpec((_D, _H), lambda i: (0, 0))
_spec_wq = pl.BlockSpec((_H, _H), lambda i: (0, 0))
_spec_bh = pl.BlockSpec((1, _H), lambda i: (0, 0))
_spec_bat = pl.BlockSpec((_RB, 1), lambda i: (i, 0))

_zs_shape = jax.ShapeDtypeStruct((_NC, _NP, _H), jnp.float32)

_mm1 = pl.pallas_call(
    _mm1_body,
    grid=(_GRID,),
    in_specs=[_spec_dp, _spec_x, _spec_wh, _spec_wh],
    out_specs=_spec_z,
    out_shape=_zs_shape,
)

_mid = pl.pallas_call(
    _mid_body,
    grid=(_GRID,),
    in_specs=[_spec_dp, _spec_pl, _spec_pr, _spec_z, _spec_bh, _spec_bh,
              _spec_wq, _spec_wq, _spec_wq, _spec_wq],
    out_specs=_spec_z,
    out_shape=_zs_shape,
)

_pool = pl.pallas_call(
    _pool_body,
    grid=(_GRID,),
    in_specs=[_spec_dp, _spec_pl, _spec_pr, _spec_z, _spec_bh, _spec_bh,
              _spec_bat],
    out_specs=pl.BlockSpec((_NC, _G, _H), lambda i: (0, 0, 0)),
    out_shape=jax.ShapeDtypeStruct((_NC, _G, _H), jnp.float32),
    scratch_shapes=[
        pltpu.VMEM((_G, _H), jnp.float32),
        pltpu.VMEM((_G, _H), jnp.float32),
        pltpu.VMEM((_G, _H), jnp.float32),
    ],
)


@jax.jit
def kernel(x, edge_index, batch, W1, b1, W2, b2, W3, b3):
    src = edge_index[0].astype(jnp.int32)
    dst = edge_index[1].astype(jnp.int32)
    pad = _EPAD - _E
    src_p = jnp.concatenate([src, jnp.zeros((pad,), jnp.int32)])
    dst_p = jnp.concatenate([dst, jnp.full((pad,), _N, jnp.int32)])
    src3 = src_p.reshape(_NS, _NCH, _CH)
    dst3 = dst_p.reshape(_NS, _NCH, _CH)
    x_p = jnp.pad(x, ((0, _NP - _N), (0, 0)))
    bat = jnp.pad(batch.astype(jnp.int32), (0, _NP - _N), constant_values=_G)
    bat = bat.reshape(_NP, 1)

    halves = lambda w: (w[:, :_H], w[:, _H:])
    quads = lambda w: (w[:_H, :_H], w[:_H, _H:], w[_H:, :_H], w[_H:, _H:])
    w1l, w1r = halves(W1)
    q2 = quads(W2)
    q3 = quads(W3)
    b1l, b1r = b1[:_H].reshape(1, _H), b1[_H:].reshape(1, _H)
    b2l, b2r = b2[:_H].reshape(1, _H), b2[_H:].reshape(1, _H)
    b3l, b3r = b3[:_H].reshape(1, _H), b3[_H:].reshape(1, _H)

    deg_parts = _deg_kernel(dst_p.reshape(_NW, _EPAD // _NW))
    z1 = _mm1(deg_parts, x_p, w1l, w1r)
    p1 = _agg_kernel(z1, src3, dst3)
    z2 = _mid(deg_parts, p1, p1, z1, b1l, b1r, *q2)
    p2 = _agg_kernel(z2, src3, dst3)
    z3 = _mid(deg_parts, p2, p2, z2, b2l, b2r, *q3)
    p3 = _agg_kernel(z3, src3, dst3)
    out = _pool(deg_parts, p3, p3, z3, b3l, b3r, bat)
    return jnp.concatenate([out[0], out[1]], axis=1)
